# baseline (speedup 1.0000x reference)
import functools

import jax
import jax.numpy as jnp
from jax import lax
from jax.experimental import pallas as pl
from jax.experimental.pallas import tpu as pltpu
from jax.experimental.pallas import tpu_sc as plsc

D = 256
KNN = 10


def _knn_body(xt_ref, xb_ref, w1_ref, h1_ref, idx_ref, *, R, B, TB, OFF):
    t = pl.program_id(0)
    base = (t % TB) * R
    boff = (t // TB) * B + OFF
    xt = xt_ref[...]
    xb = xb_ref[...]
    sq_t = jnp.sum(xt * xt, axis=1)
    sq_b = jnp.sum(xb * xb, axis=1)
    g = jnp.dot(xt, xb.T, preferred_element_type=jnp.float32)
    d2 = sq_t[:, None] + sq_b[None, :] - 2.0 * g
    row_iota = lax.broadcasted_iota(jnp.int32, (R, B), 0)
    col_iota = lax.broadcasted_iota(jnp.int32, (R, B), 1)
    d2 = jnp.where(col_iota == row_iota + base, d2 + 1e9, d2)
    adj = jnp.zeros((R, B), jnp.float32)
    js = []
    for _ in range(KNN):
        m = jnp.min(d2, axis=1)
        cand = jnp.where(d2 == m[:, None], col_iota, B)
        j = jnp.min(cand, axis=1)
        sel = col_iota == j[:, None]
        adj = adj + sel.astype(jnp.float32)
        d2 = jnp.where(sel, jnp.float32(1e30), d2)
        js.append(j)
    idx_ref[...] = jnp.stack(js, axis=1) + boff
    agg = (jnp.dot(adj, xb, preferred_element_type=jnp.float32) + xt) / 11.0
    h1_ref[...] = jnp.maximum(
        jnp.dot(agg, w1_ref[...], preferred_element_type=jnp.float32), 0.0)


def _knn(x, w1, B, R, OFF):
    N = x.shape[0]
    TB = B // R
    return pl.pallas_call(
        functools.partial(_knn_body, R=R, B=B, TB=TB, OFF=OFF),
        grid=(N // R,),
        in_specs=[
            pl.BlockSpec((R, D), lambda t: (t, 0)),
            pl.BlockSpec((B, D), lambda t: (t // TB, 0)),
            pl.BlockSpec((D, D), lambda t: (0, 0)),
        ],
        out_specs=[
            pl.BlockSpec((R, D), lambda t: (t, 0)),
            pl.BlockSpec((R, KNN), lambda t: (t, 0)),
        ],
        out_shape=[
            jax.ShapeDtypeStruct((N, D), jnp.float32),
            jax.ShapeDtypeStruct((N, KNN), jnp.int32),
        ],
    )(x, x, w1)


def _sc_agg(table, idx_flat):
    N = table.shape[0]
    NW = 32
    PER_W = N // NW
    P = 8
    STEPS = PER_W // P
    mesh = plsc.VectorSubcoreMesh(core_axis_name="c", subcore_axis_name="s",
                                  num_cores=2, num_subcores=16)

    @functools.partial(
        pl.kernel,
        mesh=mesh,
        out_type=jax.ShapeDtypeStruct((N, D), jnp.float32),
        scratch_types=[
            pltpu.VMEM((P * KNN,), jnp.int32),
            pltpu.VMEM((P * KNN, D), jnp.float32),
            pltpu.VMEM((P, D), jnp.float32),
            pltpu.SemaphoreType.DMA,
        ],
    )
    def k(tab_hbm, idx_hbm, out_hbm, idx_v, rows_v, acc_v, sem):
        wid = lax.axis_index("s") * 2 + lax.axis_index("c")
        base = wid * PER_W

        def body(st, carry):
            node0 = base + st * P
            pltpu.sync_copy(idx_hbm.at[pl.ds(node0 * KNN, P * KNN)], idx_v)
            pltpu.async_copy(tab_hbm.at[idx_v], rows_v, sem).wait()
            for p in range(P):
                for dc in range(D // 16):
                    sl = pl.ds(dc * 16, 16)
                    acc = rows_v[p * KNN, sl]
                    for j in range(1, KNN):
                        acc = acc + rows_v[p * KNN + j, sl]
                    acc_v[p, sl] = acc
            pltpu.sync_copy(acc_v, out_hbm.at[pl.ds(node0, P)])
            return carry

        lax.fori_loop(0, STEPS, body, 0)

    return k(table, idx_flat)


def _gcn2_body(x_ref, agg_ref, w_ref, hn1_ref, hn2_ref):
    x = x_ref[...]
    h = (agg_ref[...] + x) / 11.0
    h = jnp.maximum(jnp.dot(h, w_ref[...], preferred_element_type=jnp.float32),
                    0.0)
    hn1_ref[...] = x / (jnp.sqrt(jnp.sum(x * x, axis=1, keepdims=True)) + 1e-12)
    hn2_ref[...] = h / (jnp.sqrt(jnp.sum(h * h, axis=1, keepdims=True)) + 1e-12)


def _gcn2(x, agg, w, R=512):
    N = x.shape[0]
    return pl.pallas_call(
        _gcn2_body,
        grid=(N // R,),
        in_specs=[
            pl.BlockSpec((R, D), lambda t: (t, 0)),
            pl.BlockSpec((R, D), lambda t: (t, 0)),
            pl.BlockSpec((D, D), lambda t: (0, 0)),
        ],
        out_specs=[
            pl.BlockSpec((R, D), lambda t: (t, 0)),
            pl.BlockSpec((R, D), lambda t: (t, 0)),
        ],
        out_shape=[
            jax.ShapeDtypeStruct((N, D), jnp.float32),
            jax.ShapeDtypeStruct((N, D), jnp.float32),
        ],
    )(x, agg, w)


def _sims_body(q1_ref, g1_ref, q2_ref, g2_ref, s0_ref, sl_ref):
    s0_ref[...] = jnp.dot(q1_ref[...], g1_ref[...].T,
                          preferred_element_type=jnp.float32)
    sl_ref[...] = jnp.dot(q2_ref[...], g2_ref[...].T,
                          preferred_element_type=jnp.float32)


def _sims(qn1, gn1, qn2, gn2, TQ=512, TG=2048):
    NQ, NG = qn1.shape[0], gn1.shape[0]
    return pl.pallas_call(
        _sims_body,
        grid=(NQ // TQ, NG // TG),
        in_specs=[
            pl.BlockSpec((TQ, D), lambda i, j: (i, 0)),
            pl.BlockSpec((TG, D), lambda i, j: (j, 0)),
            pl.BlockSpec((TQ, D), lambda i, j: (i, 0)),
            pl.BlockSpec((TG, D), lambda i, j: (j, 0)),
        ],
        out_specs=[
            pl.BlockSpec((TQ, TG), lambda i, j: (i, j)),
            pl.BlockSpec((TQ, TG), lambda i, j: (i, j)),
        ],
        out_shape=[
            jax.ShapeDtypeStruct((NQ, NG), jnp.float32),
            jax.ShapeDtypeStruct((NQ, NG), jnp.float32),
        ],
    )(qn1, gn1, qn2, gn2)


def kernel(qf, gf, W1, W2):
    NQ = qf.shape[0]
    qh1, qidx = _knn(qf, W1, B=1024, R=256, OFF=0)
    gh1, gidx = _knn(gf, W1, B=4096, R=256, OFF=NQ)
    h1 = jnp.concatenate([qh1, gh1], axis=0)
    idx_flat = jnp.concatenate([qidx, gidx], axis=0).reshape(-1)
    agg2 = _sc_agg(h1, idx_flat)
    hn1, hn2 = _gcn2(h1, agg2, W2)
    return _sims(hn1[:NQ], hn1[NQ:], hn2[:NQ], hn2[NQ:])


# speedup vs baseline: 1.1985x; 1.1985x over previous
import functools

import jax
import jax.numpy as jnp
from jax import lax
from jax.experimental import pallas as pl
from jax.experimental.pallas import tpu as pltpu
from jax.experimental.pallas import tpu_sc as plsc

D = 256
KNN = 10


def _knn_body(xt_ref, xb_ref, w1_ref, h1_ref, idx_ref, *, R, B, TB, OFF):
    t = pl.program_id(0)
    base = (t % TB) * R
    boff = (t // TB) * B + OFF
    xt = xt_ref[...]
    xb = xb_ref[...]
    sq_t = jnp.sum(xt * xt, axis=1)
    sq_b = jnp.sum(xb * xb, axis=1)
    g = jnp.dot(xt, xb.T, preferred_element_type=jnp.float32)
    d2 = sq_t[:, None] + sq_b[None, :] - 2.0 * g
    row_iota = lax.broadcasted_iota(jnp.int32, (R, B), 0)
    col_iota = lax.broadcasted_iota(jnp.int32, (R, B), 1)
    d2 = jnp.where(col_iota == row_iota + base, d2 + 1e9, d2)
    adj = jnp.zeros((R, B), jnp.float32)
    js = []
    for _ in range(KNN):
        m = jnp.min(d2, axis=1)
        cand = jnp.where(d2 == m[:, None], col_iota, B)
        j = jnp.min(cand, axis=1)
        sel = col_iota == j[:, None]
        adj = adj + sel.astype(jnp.float32)
        d2 = jnp.where(sel, jnp.float32(1e30), d2)
        js.append(j)
    idx_ref[...] = jnp.stack(js, axis=1) + boff
    agg = (jnp.dot(adj, xb, preferred_element_type=jnp.float32) + xt) / 11.0
    h1_ref[...] = jnp.maximum(
        jnp.dot(agg, w1_ref[...], preferred_element_type=jnp.float32), 0.0)


def _knn(x, w1, B, R, OFF):
    N = x.shape[0]
    TB = B // R
    return pl.pallas_call(
        functools.partial(_knn_body, R=R, B=B, TB=TB, OFF=OFF),
        grid=(N // R,),
        in_specs=[
            pl.BlockSpec((R, D), lambda t: (t, 0)),
            pl.BlockSpec((B, D), lambda t: (t // TB, 0)),
            pl.BlockSpec((D, D), lambda t: (0, 0)),
        ],
        out_specs=[
            pl.BlockSpec((R, D), lambda t: (t, 0)),
            pl.BlockSpec((R, KNN), lambda t: (t, 0)),
        ],
        out_shape=[
            jax.ShapeDtypeStruct((N, D), jnp.float32),
            jax.ShapeDtypeStruct((N, KNN), jnp.int32),
        ],
    )(x, x, w1)


def _sc_agg(table, idx_flat):
    N = table.shape[0]
    NW = 32
    PER_W = N // NW
    P = 8
    G = P * KNN
    STEPS = PER_W // P
    mesh = plsc.VectorSubcoreMesh(core_axis_name="c", subcore_axis_name="s",
                                  num_cores=2, num_subcores=16)

    @functools.partial(
        pl.kernel,
        mesh=mesh,
        out_type=jax.ShapeDtypeStruct((N, D), jnp.float32),
        scratch_types=[
            pltpu.VMEM((PER_W * KNN,), jnp.int32),
            pltpu.VMEM((G, D), jnp.float32),
            pltpu.VMEM((G, D), jnp.float32),
            pltpu.VMEM((P, D), jnp.float32),
            pltpu.SemaphoreType.DMA,
            pltpu.SemaphoreType.DMA,
        ],
    )
    def k(tab_hbm, idx_hbm, out_hbm, idx_v, rows0, rows1, acc_v, sem0, sem1):
        wid = lax.axis_index("s") * 2 + lax.axis_index("c")
        base = wid * PER_W
        pltpu.sync_copy(idx_hbm.at[pl.ds(base * KNN, PER_W * KNN)], idx_v)
        rows = (rows0, rows1)
        sems = (sem0, sem1)

        def fire(st, b):
            off = pl.multiple_of(st * G, 8)
            pltpu.async_copy(tab_hbm.at[idx_v.at[pl.ds(off, G)]], rows[b],
                             sems[b])

        def drain(b):
            pltpu.make_async_copy(tab_hbm.at[idx_v.at[pl.ds(0, G)]], rows[b],
                                  sems[b]).wait()

        fire(0, 0)

        def gbody(gi, carry):
            for b in range(2):
                st = gi * 2 + b
                drain(b)

                @pl.when(st + 1 < STEPS)
                def _():
                    fire(st + 1, 1 - b)

                rv = rows[b]

                def pbody(p, c2):
                    r0 = p * KNN
                    for dc in range(D // 16):
                        sl = pl.ds(dc * 16, 16)
                        acc = rv[r0, sl]
                        for j in range(1, KNN):
                            acc = acc + rv[r0 + j, sl]
                        acc_v[p, sl] = acc
                    return c2

                lax.fori_loop(0, P, pbody, 0)
                node0 = pl.multiple_of(base + st * P, 8)
                pltpu.sync_copy(acc_v, out_hbm.at[pl.ds(node0, P)])
            return carry

        lax.fori_loop(0, STEPS // 2, gbody, 0)

    return k(table, idx_flat)


def _gcn2_body(x_ref, agg_ref, w_ref, hn1_ref, hn2_ref):
    x = x_ref[...]
    h = (agg_ref[...] + x) / 11.0
    h = jnp.maximum(jnp.dot(h, w_ref[...], preferred_element_type=jnp.float32),
                    0.0)
    hn1_ref[...] = x / (jnp.sqrt(jnp.sum(x * x, axis=1, keepdims=True)) + 1e-12)
    hn2_ref[...] = h / (jnp.sqrt(jnp.sum(h * h, axis=1, keepdims=True)) + 1e-12)


def _gcn2(x, agg, w, R=512):
    N = x.shape[0]
    return pl.pallas_call(
        _gcn2_body,
        grid=(N // R,),
        in_specs=[
            pl.BlockSpec((R, D), lambda t: (t, 0)),
            pl.BlockSpec((R, D), lambda t: (t, 0)),
            pl.BlockSpec((D, D), lambda t: (0, 0)),
        ],
        out_specs=[
            pl.BlockSpec((R, D), lambda t: (t, 0)),
            pl.BlockSpec((R, D), lambda t: (t, 0)),
        ],
        out_shape=[
            jax.ShapeDtypeStruct((N, D), jnp.float32),
            jax.ShapeDtypeStruct((N, D), jnp.float32),
        ],
    )(x, agg, w)


def _sims_body(q1_ref, g1_ref, q2_ref, g2_ref, s0_ref, sl_ref):
    s0_ref[...] = jnp.dot(q1_ref[...], g1_ref[...].T,
                          preferred_element_type=jnp.float32)
    sl_ref[...] = jnp.dot(q2_ref[...], g2_ref[...].T,
                          preferred_element_type=jnp.float32)


def _sims(qn1, gn1, qn2, gn2, TQ=512, TG=2048):
    NQ, NG = qn1.shape[0], gn1.shape[0]
    return pl.pallas_call(
        _sims_body,
        grid=(NQ // TQ, NG // TG),
        in_specs=[
            pl.BlockSpec((TQ, D), lambda i, j: (i, 0)),
            pl.BlockSpec((TG, D), lambda i, j: (j, 0)),
            pl.BlockSpec((TQ, D), lambda i, j: (i, 0)),
            pl.BlockSpec((TG, D), lambda i, j: (j, 0)),
        ],
        out_specs=[
            pl.BlockSpec((TQ, TG), lambda i, j: (i, j)),
            pl.BlockSpec((TQ, TG), lambda i, j: (i, j)),
        ],
        out_shape=[
            jax.ShapeDtypeStruct((NQ, NG), jnp.float32),
            jax.ShapeDtypeStruct((NQ, NG), jnp.float32),
        ],
    )(qn1, gn1, qn2, gn2)


def kernel(qf, gf, W1, W2):
    NQ = qf.shape[0]
    qh1, qidx = _knn(qf, W1, B=1024, R=256, OFF=0)
    gh1, gidx = _knn(gf, W1, B=4096, R=256, OFF=NQ)
    h1 = jnp.concatenate([qh1, gh1], axis=0)
    idx_flat = jnp.concatenate([qidx, gidx], axis=0).reshape(-1)
    agg2 = _sc_agg(h1, idx_flat)
    hn1, hn2 = _gcn2(h1, agg2, W2)
    return _sims(hn1[:NQ], hn1[NQ:], hn2[:NQ], hn2[NQ:])
